# baseline (speedup 1.0000x reference)
import functools

import jax
import jax.numpy as jnp
from jax import lax
from jax.experimental import pallas as pl
from jax.experimental.pallas import tpu as pltpu
from jax.experimental.pallas import tpu_sc as plsc

E = 64
D = 768
FF = 384
N = 2048
TM = 64
NCHUNK = N // TM
NWI = N // TM + E
Q = 6
NSTEP = NWI // Q
P = NWI * TM
NWORKER = 32
ROWS_W = N // NWORKER


def _router_body(x_ref, wg_ref, pos_ref, wie_ref, wit_ref, oh_scr, run_scr):
    x = x_ref[...]
    logits = lax.dot_general(x, wg_ref[...], (((1,), (1,)), ((), ())),
                             preferred_element_type=jnp.float32)
    iota_e = lax.broadcasted_iota(jnp.int32, (N, E), 1)
    m = jnp.max(logits, axis=1, keepdims=True)
    sel = jnp.min(jnp.where(logits == m, iota_e, E), axis=1, keepdims=True)
    oh = (iota_e == sel).astype(jnp.float32)
    oh_scr[...] = oh

    tril_s = (lax.broadcasted_iota(jnp.int32, (TM, TM), 0)
              > lax.broadcasted_iota(jnp.int32, (TM, TM), 1)).astype(jnp.float32)

    def body_a(c, running):
        ohc = oh_scr[pl.ds(c * TM, TM), :]
        run_scr[pl.ds(c, 1), :] = running
        return running + jnp.sum(ohc, axis=0, keepdims=True)

    counts = lax.fori_loop(0, NCHUNK, body_a, jnp.zeros((1, E), jnp.float32))

    tiles = jnp.ceil(counts * (1.0 / TM))
    su = (lax.broadcasted_iota(jnp.int32, (E, E), 0)
          < lax.broadcasted_iota(jnp.int32, (E, E), 1)).astype(jnp.float32)
    tstart = lax.dot_general(tiles, su, (((1,), (0,)), ((), ())),
                             preferred_element_type=jnp.float32)
    total = jnp.sum(tiles)
    aoff = tstart * TM

    def body_b(c, carry):
        ohc = oh_scr[pl.ds(c * TM, TM), :]
        excl = lax.dot_general(tril_s, ohc, (((1,), (0,)), ((), ())),
                               preferred_element_type=jnp.float32)
        base = run_scr[pl.ds(c, 1), :] + aoff
        pos_c = jnp.sum((excl + base) * ohc, axis=1, keepdims=True)
        pos_ref[pl.ds(c * TM, TM), :] = pos_c.astype(jnp.int32)
        return carry

    lax.fori_loop(0, NCHUNK, body_b, 0)

    j = lax.broadcasted_iota(jnp.int32, (128, 1), 0).astype(jnp.float32)
    jj = jnp.minimum(j, total - 1.0)
    e_j = jnp.sum((tstart <= jj).astype(jnp.float32), axis=1, keepdims=True) - 1.0
    wie_ref[...] = e_j.astype(jnp.int32)
    wit_ref[...] = jnp.where(j < total, j, float(NWI)).astype(jnp.int32)


def _route(x, wg):
    return pl.pallas_call(
        _router_body,
        out_shape=[jax.ShapeDtypeStruct((N, 1), jnp.int32),
                   jax.ShapeDtypeStruct((128, 1), jnp.int32),
                   jax.ShapeDtypeStruct((128, 1), jnp.int32)],
        scratch_shapes=[pltpu.VMEM((N, E), jnp.float32),
                        pltpu.VMEM((NCHUNK, E), jnp.float32)],
    )(x, wg)


def _ffn_body(wie_ref, xs_ref, *rest):
    ys_ref = rest[-1]
    for q in range(Q):
        w1_ref, w3_ref, w2_ref = rest[q], rest[Q + q], rest[2 * Q + q]
        xb = xs_ref[pl.ds(q * TM, TM), :]
        a1 = lax.dot_general(xb, w1_ref[0], (((1,), (1,)), ((), ())),
                             preferred_element_type=jnp.float32)
        a3 = lax.dot_general(xb, w3_ref[0], (((1,), (1,)), ((), ())),
                             preferred_element_type=jnp.float32)
        h = (a1 / (1.0 + jnp.exp(-a1))) * a3
        ys_ref[pl.ds(q * TM, TM), :] = lax.dot_general(
            h, w2_ref[0], (((1,), (1,)), ((), ())),
            preferred_element_type=jnp.float32)


def _ffn(x_s, w1, w3, w2, wie):
    w_specs = []
    for shape in ((1, FF, D), (1, FF, D), (1, D, FF)):
        for q in range(Q):
            w_specs.append(pl.BlockSpec(
                shape, lambda i, wie, q=q: (wie[Q * i + q], 0, 0)))
    grid_spec = pltpu.PrefetchScalarGridSpec(
        num_scalar_prefetch=1,
        grid=(NSTEP,),
        in_specs=[pl.BlockSpec((Q * TM, D), lambda i, wie: (i, 0))] + w_specs,
        out_specs=pl.BlockSpec((Q * TM, D), lambda i, wie: (i, 0)),
    )
    return pl.pallas_call(
        _ffn_body, grid_spec=grid_spec,
        out_shape=jax.ShapeDtypeStruct((P, D), jnp.float32),
    )(wie, x_s, *([w1] * Q), *([w3] * Q), *([w2] * Q))


def _sc_mesh():
    return plsc.VectorSubcoreMesh(core_axis_name="c", subcore_axis_name="s")


SC_CH = 4
CH = ROWS_W // SC_CH


def _sc_scatter(x, pos2d):
    @functools.partial(
        pl.kernel, mesh=_sc_mesh(),
        out_type=jax.ShapeDtypeStruct((P, D), jnp.float32),
        scratch_types=[pltpu.VMEM((SC_CH, CH), jnp.int32),
                       pltpu.VMEM((SC_CH, CH, D), jnp.float32),
                       pltpu.SemaphoreType.DMA,
                       pltpu.SemaphoreType.DMA,
                       pltpu.SemaphoreType.DMA,
                       pltpu.SemaphoreType.DMA,
                       pltpu.SemaphoreType.DMA,
                       pltpu.SemaphoreType.DMA],
    )
    def k(x_hbm, pos_hbm, out_hbm, idx_v, rows_v, isem, s0, s1, s2, s3, ssem):
        wid = lax.axis_index("s") * 2 + lax.axis_index("c")
        base = wid * ROWS_W
        pltpu.async_copy(pos_hbm.at[pl.ds(wid * SC_CH, SC_CH)], idx_v, isem).wait()
        lsems = [s0, s1, s2, s3]
        loads = [pltpu.async_copy(x_hbm.at[pl.ds(base + c * CH, CH)],
                                  rows_v.at[c], lsems[c]) for c in range(SC_CH)]
        scats = []
        for c in range(SC_CH):
            loads[c].wait()
            scats.append(pltpu.async_copy(rows_v.at[c], out_hbm.at[idx_v.at[c]],
                                          ssem))
        for s in scats:
            s.wait()

    return k(x, pos2d)


def _sc_gather(y_s, pos2d):
    @functools.partial(
        pl.kernel, mesh=_sc_mesh(),
        out_type=jax.ShapeDtypeStruct((N, D), jnp.float32),
        scratch_types=[pltpu.VMEM((SC_CH, CH), jnp.int32),
                       pltpu.VMEM((SC_CH, CH, D), jnp.float32),
                       pltpu.SemaphoreType.DMA,
                       pltpu.SemaphoreType.DMA,
                       pltpu.SemaphoreType.DMA,
                       pltpu.SemaphoreType.DMA,
                       pltpu.SemaphoreType.DMA,
                       pltpu.SemaphoreType.DMA],
    )
    def k(ys_hbm, pos_hbm, out_hbm, idx_v, rows_v, isem, s0, s1, s2, s3, osem):
        wid = lax.axis_index("s") * 2 + lax.axis_index("c")
        base = wid * ROWS_W
        pltpu.async_copy(pos_hbm.at[pl.ds(wid * SC_CH, SC_CH)], idx_v, isem).wait()
        gsems = [s0, s1, s2, s3]
        gathers = [pltpu.async_copy(ys_hbm.at[idx_v.at[c]], rows_v.at[c],
                                    gsems[c]) for c in range(SC_CH)]
        stores = []
        for c in range(SC_CH):
            gathers[c].wait()
            stores.append(pltpu.async_copy(
                rows_v.at[c], out_hbm.at[pl.ds(base + c * CH, CH)], osem))
        for s in stores:
            s.wait()

    return k(y_s, pos2d)


def kernel(hidden_states, Wg, W1, W3, W2):
    b, s, d = hidden_states.shape
    x = hidden_states.reshape(b * s, d)
    pos2d, wie2d, wit2d = _route(x, Wg)
    posc = pos2d.reshape(N // CH, CH)
    wie = wie2d.reshape(128)[:NWI]
    del wit2d
    x_s = _sc_scatter(x, posc)
    y_s = _ffn(x_s, W1, W3, W2, wie)
    out = _sc_gather(y_s, posc)
    return out.reshape(b, s, d)


# speedup vs baseline: 1.0507x; 1.0507x over previous
import functools

import jax
import jax.numpy as jnp
from jax import lax
from jax.experimental import pallas as pl
from jax.experimental.pallas import tpu as pltpu
from jax.experimental.pallas import tpu_sc as plsc

E = 64
D = 768
FF = 384
N = 2048
TM = 128
NCHUNK = N // TM
NWI = N // TM + E
Q = 5
NSTEP = NWI // Q
P = NWI * TM
NWORKER = 32
ROWS_W = N // NWORKER


def _router_body(x_ref, wg_ref, pos_ref, wie_ref, wit_ref, oh_scr, run_scr):
    x = x_ref[...]
    logits = lax.dot_general(x, wg_ref[...], (((1,), (1,)), ((), ())),
                             preferred_element_type=jnp.float32)
    iota_e = lax.broadcasted_iota(jnp.int32, (N, E), 1)
    m = jnp.max(logits, axis=1, keepdims=True)
    sel = jnp.min(jnp.where(logits == m, iota_e, E), axis=1, keepdims=True)
    oh = (iota_e == sel).astype(jnp.float32)
    oh_scr[...] = oh

    tril_s = (lax.broadcasted_iota(jnp.int32, (TM, TM), 0)
              > lax.broadcasted_iota(jnp.int32, (TM, TM), 1)).astype(jnp.float32)

    def body_a(c, running):
        ohc = oh_scr[pl.ds(c * TM, TM), :]
        run_scr[pl.ds(c, 1), :] = running
        return running + jnp.sum(ohc, axis=0, keepdims=True)

    counts = lax.fori_loop(0, NCHUNK, body_a, jnp.zeros((1, E), jnp.float32))

    tiles = jnp.ceil(counts * (1.0 / TM))
    su = (lax.broadcasted_iota(jnp.int32, (E, E), 0)
          < lax.broadcasted_iota(jnp.int32, (E, E), 1)).astype(jnp.float32)
    tstart = lax.dot_general(tiles, su, (((1,), (0,)), ((), ())),
                             preferred_element_type=jnp.float32)
    total = jnp.sum(tiles)
    aoff = tstart * TM

    def body_b(c, carry):
        ohc = oh_scr[pl.ds(c * TM, TM), :]
        excl = lax.dot_general(tril_s, ohc, (((1,), (0,)), ((), ())),
                               preferred_element_type=jnp.float32)
        base = run_scr[pl.ds(c, 1), :] + aoff
        pos_c = jnp.sum((excl + base) * ohc, axis=1, keepdims=True)
        pos_ref[pl.ds(c * TM, TM), :] = pos_c.astype(jnp.int32)
        return carry

    lax.fori_loop(0, NCHUNK, body_b, 0)

    j = lax.broadcasted_iota(jnp.int32, (128, 1), 0).astype(jnp.float32)
    jj = jnp.minimum(j, total - 1.0)
    e_j = jnp.sum((tstart <= jj).astype(jnp.float32), axis=1, keepdims=True) - 1.0
    wie_ref[...] = e_j.astype(jnp.int32)
    wit_ref[...] = jnp.where(j < total, j, float(NWI)).astype(jnp.int32)


def _route(x, wg):
    return pl.pallas_call(
        _router_body,
        out_shape=[jax.ShapeDtypeStruct((N, 1), jnp.int32),
                   jax.ShapeDtypeStruct((128, 1), jnp.int32),
                   jax.ShapeDtypeStruct((128, 1), jnp.int32)],
        scratch_shapes=[pltpu.VMEM((N, E), jnp.float32),
                        pltpu.VMEM((NCHUNK, E), jnp.float32)],
    )(x, wg)


def _ffn_body(wie_ref, xs_ref, *rest):
    ys_ref = rest[-1]
    for q in range(Q):
        w1_ref, w3_ref, w2_ref = rest[q], rest[Q + q], rest[2 * Q + q]
        xb = xs_ref[pl.ds(q * TM, TM), :]
        a1 = lax.dot_general(xb, w1_ref[0], (((1,), (1,)), ((), ())),
                             preferred_element_type=jnp.float32)
        a3 = lax.dot_general(xb, w3_ref[0], (((1,), (1,)), ((), ())),
                             preferred_element_type=jnp.float32)
        h = (a1 / (1.0 + jnp.exp(-a1))) * a3
        ys_ref[pl.ds(q * TM, TM), :] = lax.dot_general(
            h, w2_ref[0], (((1,), (1,)), ((), ())),
            preferred_element_type=jnp.float32)


def _ffn(x_s, w1, w3, w2, wie):
    w_specs = []
    for shape in ((1, FF, D), (1, FF, D), (1, D, FF)):
        for q in range(Q):
            w_specs.append(pl.BlockSpec(
                shape, lambda i, wie, q=q: (wie[Q * i + q], 0, 0)))
    grid_spec = pltpu.PrefetchScalarGridSpec(
        num_scalar_prefetch=1,
        grid=(NSTEP,),
        in_specs=[pl.BlockSpec((Q * TM, D), lambda i, wie: (i, 0))] + w_specs,
        out_specs=pl.BlockSpec((Q * TM, D), lambda i, wie: (i, 0)),
    )
    return pl.pallas_call(
        _ffn_body, grid_spec=grid_spec,
        out_shape=jax.ShapeDtypeStruct((P, D), jnp.float32),
    )(wie, x_s, *([w1] * Q), *([w3] * Q), *([w2] * Q))


def _sc_mesh():
    return plsc.VectorSubcoreMesh(core_axis_name="c", subcore_axis_name="s")


SC_CH = 4
CH = ROWS_W // SC_CH


def _sc_scatter(x, pos2d):
    @functools.partial(
        pl.kernel, mesh=_sc_mesh(),
        out_type=jax.ShapeDtypeStruct((P, D), jnp.float32),
        scratch_types=[pltpu.VMEM((SC_CH, CH), jnp.int32),
                       pltpu.VMEM((SC_CH, CH, D), jnp.float32),
                       pltpu.SemaphoreType.DMA,
                       pltpu.SemaphoreType.DMA,
                       pltpu.SemaphoreType.DMA,
                       pltpu.SemaphoreType.DMA,
                       pltpu.SemaphoreType.DMA,
                       pltpu.SemaphoreType.DMA],
    )
    def k(x_hbm, pos_hbm, out_hbm, idx_v, rows_v, isem, s0, s1, s2, s3, ssem):
        wid = lax.axis_index("s") * 2 + lax.axis_index("c")
        base = wid * ROWS_W
        pltpu.async_copy(pos_hbm.at[pl.ds(wid * SC_CH, SC_CH)], idx_v, isem).wait()
        lsems = [s0, s1, s2, s3]
        loads = [pltpu.async_copy(x_hbm.at[pl.ds(base + c * CH, CH)],
                                  rows_v.at[c], lsems[c]) for c in range(SC_CH)]
        scats = []
        for c in range(SC_CH):
            loads[c].wait()
            scats.append(pltpu.async_copy(rows_v.at[c], out_hbm.at[idx_v.at[c]],
                                          ssem))
        for s in scats:
            s.wait()

    return k(x, pos2d)


def _sc_gather(y_s, pos2d):
    @functools.partial(
        pl.kernel, mesh=_sc_mesh(),
        out_type=jax.ShapeDtypeStruct((N, D), jnp.float32),
        scratch_types=[pltpu.VMEM((SC_CH, CH), jnp.int32),
                       pltpu.VMEM((SC_CH, CH, D), jnp.float32),
                       pltpu.SemaphoreType.DMA,
                       pltpu.SemaphoreType.DMA,
                       pltpu.SemaphoreType.DMA,
                       pltpu.SemaphoreType.DMA,
                       pltpu.SemaphoreType.DMA,
                       pltpu.SemaphoreType.DMA],
    )
    def k(ys_hbm, pos_hbm, out_hbm, idx_v, rows_v, isem, s0, s1, s2, s3, osem):
        wid = lax.axis_index("s") * 2 + lax.axis_index("c")
        base = wid * ROWS_W
        pltpu.async_copy(pos_hbm.at[pl.ds(wid * SC_CH, SC_CH)], idx_v, isem).wait()
        gsems = [s0, s1, s2, s3]
        gathers = [pltpu.async_copy(ys_hbm.at[idx_v.at[c]], rows_v.at[c],
                                    gsems[c]) for c in range(SC_CH)]
        stores = []
        for c in range(SC_CH):
            gathers[c].wait()
            stores.append(pltpu.async_copy(
                rows_v.at[c], out_hbm.at[pl.ds(base + c * CH, CH)], osem))
        for s in stores:
            s.wait()

    return k(y_s, pos2d)


def kernel(hidden_states, Wg, W1, W3, W2):
    b, s, d = hidden_states.shape
    x = hidden_states.reshape(b * s, d)
    pos2d, wie2d, wit2d = _route(x, Wg)
    posc = pos2d.reshape(N // CH, CH)
    wie = wie2d.reshape(128)[:NWI]
    del wit2d
    x_s = _sc_scatter(x, posc)
    y_s = _ffn(x_s, W1, W3, W2, wie)
    out = _sc_gather(y_s, posc)
    return out.reshape(b, s, d)


# speedup vs baseline: 1.0597x; 1.0086x over previous
import functools

import jax
import jax.numpy as jnp
from jax import lax
from jax.experimental import pallas as pl
from jax.experimental.pallas import tpu as pltpu
from jax.experimental.pallas import tpu_sc as plsc

E = 64
D = 768
FF = 384
N = 2048
TM = 128
NCHUNK = N // TM
NWI = N // TM + E
Q = 4
NSTEP = NWI // Q
P = NWI * TM
NWORKER = 32
ROWS_W = N // NWORKER


def _router_body(x_ref, wg_ref, pos_ref, wie_ref, wit_ref, oh_scr, run_scr):
    x = x_ref[...]
    logits = lax.dot_general(x, wg_ref[...], (((1,), (1,)), ((), ())),
                             preferred_element_type=jnp.float32)
    iota_e = lax.broadcasted_iota(jnp.int32, (N, E), 1)
    m = jnp.max(logits, axis=1, keepdims=True)
    sel = jnp.min(jnp.where(logits == m, iota_e, E), axis=1, keepdims=True)
    oh = (iota_e == sel).astype(jnp.float32)
    oh_scr[...] = oh

    tril_s = (lax.broadcasted_iota(jnp.int32, (TM, TM), 0)
              > lax.broadcasted_iota(jnp.int32, (TM, TM), 1)).astype(jnp.float32)

    def body_a(c, running):
        ohc = oh_scr[pl.ds(c * TM, TM), :]
        run_scr[pl.ds(c, 1), :] = running
        return running + jnp.sum(ohc, axis=0, keepdims=True)

    counts = lax.fori_loop(0, NCHUNK, body_a, jnp.zeros((1, E), jnp.float32))

    tiles = jnp.ceil(counts * (1.0 / TM))
    su = (lax.broadcasted_iota(jnp.int32, (E, E), 0)
          < lax.broadcasted_iota(jnp.int32, (E, E), 1)).astype(jnp.float32)
    tstart = lax.dot_general(tiles, su, (((1,), (0,)), ((), ())),
                             preferred_element_type=jnp.float32)
    total = jnp.sum(tiles)
    aoff = tstart * TM

    def body_b(c, carry):
        ohc = oh_scr[pl.ds(c * TM, TM), :]
        excl = lax.dot_general(tril_s, ohc, (((1,), (0,)), ((), ())),
                               preferred_element_type=jnp.float32)
        base = run_scr[pl.ds(c, 1), :] + aoff
        pos_c = jnp.sum((excl + base) * ohc, axis=1, keepdims=True)
        pos_ref[pl.ds(c * TM, TM), :] = pos_c.astype(jnp.int32)
        return carry

    lax.fori_loop(0, NCHUNK, body_b, 0)

    j = lax.broadcasted_iota(jnp.int32, (128, 1), 0).astype(jnp.float32)
    jj = jnp.minimum(j, total - 1.0)
    e_j = jnp.sum((tstart <= jj).astype(jnp.float32), axis=1, keepdims=True) - 1.0
    wie_ref[...] = e_j.astype(jnp.int32)
    wit_ref[...] = jnp.where(j < total, j, float(NWI)).astype(jnp.int32)


def _route(x, wg):
    return pl.pallas_call(
        _router_body,
        out_shape=[jax.ShapeDtypeStruct((N, 1), jnp.int32),
                   jax.ShapeDtypeStruct((128, 1), jnp.int32),
                   jax.ShapeDtypeStruct((128, 1), jnp.int32)],
        scratch_shapes=[pltpu.VMEM((N, E), jnp.float32),
                        pltpu.VMEM((NCHUNK, E), jnp.float32)],
    )(x, wg)


def _ffn_body(wie_ref, xs_ref, *rest):
    ys_ref = rest[-1]
    for q in range(Q):
        w1_ref, w3_ref, w2_ref = rest[q], rest[Q + q], rest[2 * Q + q]
        xb = xs_ref[pl.ds(q * TM, TM), :]
        a1 = lax.dot_general(xb, w1_ref[0], (((1,), (1,)), ((), ())),
                             preferred_element_type=jnp.float32)
        a3 = lax.dot_general(xb, w3_ref[0], (((1,), (1,)), ((), ())),
                             preferred_element_type=jnp.float32)
        h = (a1 / (1.0 + jnp.exp(-a1))) * a3
        ys_ref[pl.ds(q * TM, TM), :] = lax.dot_general(
            h, w2_ref[0], (((1,), (1,)), ((), ())),
            preferred_element_type=jnp.float32)


def _ffn(x_s, w1, w3, w2, wie):
    w_specs = []
    for shape in ((1, FF, D), (1, FF, D), (1, D, FF)):
        for q in range(Q):
            w_specs.append(pl.BlockSpec(
                shape, lambda i, wie, q=q: (wie[Q * i + q], 0, 0)))
    grid_spec = pltpu.PrefetchScalarGridSpec(
        num_scalar_prefetch=1,
        grid=(NSTEP,),
        in_specs=[pl.BlockSpec((Q * TM, D), lambda i, wie: (i, 0))] + w_specs,
        out_specs=pl.BlockSpec((Q * TM, D), lambda i, wie: (i, 0)),
    )
    return pl.pallas_call(
        _ffn_body, grid_spec=grid_spec,
        out_shape=jax.ShapeDtypeStruct((P, D), jnp.float32),
    )(wie, x_s, *([w1] * Q), *([w3] * Q), *([w2] * Q))


def _sc_mesh():
    return plsc.VectorSubcoreMesh(core_axis_name="c", subcore_axis_name="s")


SC_CH = 4
CH = ROWS_W // SC_CH


def _sc_scatter(x, pos2d):
    @functools.partial(
        pl.kernel, mesh=_sc_mesh(),
        out_type=jax.ShapeDtypeStruct((P, D), jnp.float32),
        scratch_types=[pltpu.VMEM((SC_CH, CH), jnp.int32),
                       pltpu.VMEM((SC_CH, CH, D), jnp.float32),
                       pltpu.SemaphoreType.DMA,
                       pltpu.SemaphoreType.DMA,
                       pltpu.SemaphoreType.DMA,
                       pltpu.SemaphoreType.DMA,
                       pltpu.SemaphoreType.DMA,
                       pltpu.SemaphoreType.DMA],
    )
    def k(x_hbm, pos_hbm, out_hbm, idx_v, rows_v, isem, s0, s1, s2, s3, ssem):
        wid = lax.axis_index("s") * 2 + lax.axis_index("c")
        base = wid * ROWS_W
        pltpu.async_copy(pos_hbm.at[pl.ds(wid * SC_CH, SC_CH)], idx_v, isem).wait()
        lsems = [s0, s1, s2, s3]
        loads = [pltpu.async_copy(x_hbm.at[pl.ds(base + c * CH, CH)],
                                  rows_v.at[c], lsems[c]) for c in range(SC_CH)]
        scats = []
        for c in range(SC_CH):
            loads[c].wait()
            scats.append(pltpu.async_copy(rows_v.at[c], out_hbm.at[idx_v.at[c]],
                                          ssem))
        for s in scats:
            s.wait()

    return k(x, pos2d)


def _sc_gather(y_s, pos2d):
    @functools.partial(
        pl.kernel, mesh=_sc_mesh(),
        out_type=jax.ShapeDtypeStruct((N, D), jnp.float32),
        scratch_types=[pltpu.VMEM((SC_CH, CH), jnp.int32),
                       pltpu.VMEM((SC_CH, CH, D), jnp.float32),
                       pltpu.SemaphoreType.DMA,
                       pltpu.SemaphoreType.DMA,
                       pltpu.SemaphoreType.DMA,
                       pltpu.SemaphoreType.DMA,
                       pltpu.SemaphoreType.DMA,
                       pltpu.SemaphoreType.DMA],
    )
    def k(ys_hbm, pos_hbm, out_hbm, idx_v, rows_v, isem, s0, s1, s2, s3, osem):
        wid = lax.axis_index("s") * 2 + lax.axis_index("c")
        base = wid * ROWS_W
        pltpu.async_copy(pos_hbm.at[pl.ds(wid * SC_CH, SC_CH)], idx_v, isem).wait()
        gsems = [s0, s1, s2, s3]
        gathers = [pltpu.async_copy(ys_hbm.at[idx_v.at[c]], rows_v.at[c],
                                    gsems[c]) for c in range(SC_CH)]
        stores = []
        for c in range(SC_CH):
            gathers[c].wait()
            stores.append(pltpu.async_copy(
                rows_v.at[c], out_hbm.at[pl.ds(base + c * CH, CH)], osem))
        for s in stores:
            s.wait()

    return k(y_s, pos2d)


def kernel(hidden_states, Wg, W1, W3, W2):
    b, s, d = hidden_states.shape
    x = hidden_states.reshape(b * s, d)
    pos2d, wie2d, wit2d = _route(x, Wg)
    posc = pos2d.reshape(N // CH, CH)
    wie = wie2d.reshape(128)[:NWI]
    del wit2d
    x_s = _sc_scatter(x, posc)
    y_s = _ffn(x_s, W1, W3, W2, wie)
    out = _sc_gather(y_s, posc)
    return out.reshape(b, s, d)


# speedup vs baseline: 1.0601x; 1.0003x over previous
import functools

import jax
import jax.numpy as jnp
from jax import lax
from jax.experimental import pallas as pl
from jax.experimental.pallas import tpu as pltpu
from jax.experimental.pallas import tpu_sc as plsc

E = 64
D = 768
FF = 384
N = 2048
TM = 128
NCHUNK = N // TM
NWI = N // TM + E
Q = 4
NSTEP = NWI // Q
P = NWI * TM
NWORKER = 32
ROWS_W = N // NWORKER


def _router_body(x_ref, wg_ref, pos_ref, wie_ref, wit_ref, oh_scr, run_scr):
    x = x_ref[...]
    logits = lax.dot_general(x, wg_ref[...], (((1,), (1,)), ((), ())),
                             preferred_element_type=jnp.float32)
    iota_e = lax.broadcasted_iota(jnp.int32, (N, E), 1)
    m = jnp.max(logits, axis=1, keepdims=True)
    sel = jnp.min(jnp.where(logits == m, iota_e, E), axis=1, keepdims=True)
    oh = (iota_e == sel).astype(jnp.float32)
    oh_scr[...] = oh

    tril_s = (lax.broadcasted_iota(jnp.int32, (TM, TM), 0)
              > lax.broadcasted_iota(jnp.int32, (TM, TM), 1)).astype(jnp.float32)

    def body_a(c, running):
        ohc = oh_scr[pl.ds(c * TM, TM), :]
        run_scr[pl.ds(c, 1), :] = running
        return running + jnp.sum(ohc, axis=0, keepdims=True)

    counts = lax.fori_loop(0, NCHUNK, body_a, jnp.zeros((1, E), jnp.float32))

    tiles = jnp.ceil(counts * (1.0 / TM))
    su = (lax.broadcasted_iota(jnp.int32, (E, E), 0)
          < lax.broadcasted_iota(jnp.int32, (E, E), 1)).astype(jnp.float32)
    tstart = lax.dot_general(tiles, su, (((1,), (0,)), ((), ())),
                             preferred_element_type=jnp.float32)
    total = jnp.sum(tiles)
    aoff = tstart * TM

    def body_b(c, carry):
        ohc = oh_scr[pl.ds(c * TM, TM), :]
        excl = lax.dot_general(tril_s, ohc, (((1,), (0,)), ((), ())),
                               preferred_element_type=jnp.float32)
        base = run_scr[pl.ds(c, 1), :] + aoff
        pos_c = jnp.sum((excl + base) * ohc, axis=1, keepdims=True)
        pos_ref[pl.ds(c * TM, TM), :] = pos_c.astype(jnp.int32)
        return carry

    lax.fori_loop(0, NCHUNK, body_b, 0)

    j = lax.broadcasted_iota(jnp.int32, (128, 1), 0).astype(jnp.float32)
    jj = jnp.minimum(j, total - 1.0)
    e_j = jnp.sum((tstart <= jj).astype(jnp.float32), axis=1, keepdims=True) - 1.0
    wie_ref[...] = e_j.astype(jnp.int32)
    wit_ref[...] = jnp.where(j < total, j, float(NWI)).astype(jnp.int32)


def _route(x, wg):
    return pl.pallas_call(
        _router_body,
        out_shape=[jax.ShapeDtypeStruct((N, 1), jnp.int32),
                   jax.ShapeDtypeStruct((128, 1), jnp.int32),
                   jax.ShapeDtypeStruct((128, 1), jnp.int32)],
        scratch_shapes=[pltpu.VMEM((N, E), jnp.float32),
                        pltpu.VMEM((NCHUNK, E), jnp.float32)],
    )(x, wg)


def _ffn_body(wie_ref, xs_ref, *rest):
    ys_ref = rest[-1]
    for q in range(Q):
        w1_ref, w3_ref, w2_ref = rest[q], rest[Q + q], rest[2 * Q + q]
        xb = xs_ref[pl.ds(q * TM, TM), :]
        a1 = lax.dot_general(xb, w1_ref[0], (((1,), (1,)), ((), ())),
                             preferred_element_type=jnp.float32)
        a3 = lax.dot_general(xb, w3_ref[0], (((1,), (1,)), ((), ())),
                             preferred_element_type=jnp.float32)
        h = (a1 / (1.0 + jnp.exp(-a1))) * a3
        ys_ref[pl.ds(q * TM, TM), :] = lax.dot_general(
            h, w2_ref[0], (((1,), (1,)), ((), ())),
            preferred_element_type=jnp.float32)


def _ffn(x_s, w1, w3, w2, wie):
    w_specs = []
    for shape in ((1, FF, D), (1, FF, D), (1, D, FF)):
        for q in range(Q):
            w_specs.append(pl.BlockSpec(
                shape, lambda i, wie, q=q: (wie[Q * i + q], 0, 0)))
    grid_spec = pltpu.PrefetchScalarGridSpec(
        num_scalar_prefetch=1,
        grid=(NSTEP,),
        in_specs=[pl.BlockSpec((Q * TM, D), lambda i, wie: (i, 0))] + w_specs,
        out_specs=pl.BlockSpec((Q * TM, D), lambda i, wie: (i, 0)),
    )
    return pl.pallas_call(
        _ffn_body, grid_spec=grid_spec,
        out_shape=jax.ShapeDtypeStruct((P, D), jnp.float32),
    )(wie, x_s, *([w1] * Q), *([w3] * Q), *([w2] * Q))


def _sc_mesh():
    return plsc.VectorSubcoreMesh(core_axis_name="c", subcore_axis_name="s")


SC_CH = 2
CH = ROWS_W // SC_CH


def _sc_scatter(x, pos2d):
    @functools.partial(
        pl.kernel, mesh=_sc_mesh(),
        out_type=jax.ShapeDtypeStruct((P, D), jnp.float32),
        scratch_types=[pltpu.VMEM((SC_CH, CH), jnp.int32),
                       pltpu.VMEM((SC_CH, CH, D), jnp.float32),
                       pltpu.SemaphoreType.DMA,
                       pltpu.SemaphoreType.DMA,
                       pltpu.SemaphoreType.DMA,
                       pltpu.SemaphoreType.DMA,
                       pltpu.SemaphoreType.DMA,
                       pltpu.SemaphoreType.DMA],
    )
    def k(x_hbm, pos_hbm, out_hbm, idx_v, rows_v, isem, s0, s1, s2, s3, ssem):
        wid = lax.axis_index("s") * 2 + lax.axis_index("c")
        base = wid * ROWS_W
        pltpu.async_copy(pos_hbm.at[pl.ds(wid * SC_CH, SC_CH)], idx_v, isem).wait()
        lsems = [s0, s1, s2, s3]
        loads = [pltpu.async_copy(x_hbm.at[pl.ds(base + c * CH, CH)],
                                  rows_v.at[c], lsems[c]) for c in range(SC_CH)]
        scats = []
        for c in range(SC_CH):
            loads[c].wait()
            scats.append(pltpu.async_copy(rows_v.at[c], out_hbm.at[idx_v.at[c]],
                                          ssem))
        for s in scats:
            s.wait()

    return k(x, pos2d)


def _sc_gather(y_s, pos2d):
    @functools.partial(
        pl.kernel, mesh=_sc_mesh(),
        out_type=jax.ShapeDtypeStruct((N, D), jnp.float32),
        scratch_types=[pltpu.VMEM((SC_CH, CH), jnp.int32),
                       pltpu.VMEM((SC_CH, CH, D), jnp.float32),
                       pltpu.SemaphoreType.DMA,
                       pltpu.SemaphoreType.DMA,
                       pltpu.SemaphoreType.DMA,
                       pltpu.SemaphoreType.DMA,
                       pltpu.SemaphoreType.DMA,
                       pltpu.SemaphoreType.DMA],
    )
    def k(ys_hbm, pos_hbm, out_hbm, idx_v, rows_v, isem, s0, s1, s2, s3, osem):
        wid = lax.axis_index("s") * 2 + lax.axis_index("c")
        base = wid * ROWS_W
        pltpu.async_copy(pos_hbm.at[pl.ds(wid * SC_CH, SC_CH)], idx_v, isem).wait()
        gsems = [s0, s1, s2, s3]
        gathers = [pltpu.async_copy(ys_hbm.at[idx_v.at[c]], rows_v.at[c],
                                    gsems[c]) for c in range(SC_CH)]
        stores = []
        for c in range(SC_CH):
            gathers[c].wait()
            stores.append(pltpu.async_copy(
                rows_v.at[c], out_hbm.at[pl.ds(base + c * CH, CH)], osem))
        for s in stores:
            s.wait()

    return k(y_s, pos2d)


def kernel(hidden_states, Wg, W1, W3, W2):
    b, s, d = hidden_states.shape
    x = hidden_states.reshape(b * s, d)
    pos2d, wie2d, wit2d = _route(x, Wg)
    posc = pos2d.reshape(N // CH, CH)
    wie = wie2d.reshape(128)[:NWI]
    del wit2d
    x_s = _sc_scatter(x, posc)
    y_s = _ffn(x_s, W1, W3, W2, wie)
    out = _sc_gather(y_s, posc)
    return out.reshape(b, s, d)
